# baseline (speedup 1.0000x reference)
import functools

import jax
import jax.numpy as jnp
from jax import lax
from jax.experimental import pallas as pl
from jax.experimental.pallas import tpu as pltpu
from jax.experimental.pallas import tpu_sc as plsc

NUM_CORES = 2
NUM_SUBCORES = 16
NW = NUM_CORES * NUM_SUBCORES
L = 16

IN_ROWS = 8192
OUT_ROWS = 32768
D = 1024
ROWS_PER_W = OUT_ROWS // NW
MAX_REP = 3

OUT_BLOCK = 1024
N_BLOCKS = OUT_ROWS // OUT_BLOCK


def _idx_body(rt_hbm, cum_hbm, lim_hbm, idx_hbm, rt_v, cum_v, idx_v, lim_v):
    wid = lax.axis_index("s") * NUM_CORES + lax.axis_index("c")
    base = wid * ROWS_PER_W

    pltpu.sync_copy(rt_hbm, rt_v)
    pltpu.sync_copy(cum_hbm, cum_v)
    pltpu.sync_copy(lim_hbm, lim_v)
    limit = lim_v[...]

    def init(j, carry):
        idx_v[pl.ds(j * L, L)] = jnp.full((L,), IN_ROWS, jnp.int32)
        return carry

    lax.fori_loop(0, ROWS_PER_W // L, init, 0)

    lane = lax.iota(jnp.int32, L)

    def scan(j, carry):
        r = rt_v[pl.ds(j * L, L)]
        off = cum_v[pl.ds(j * L, L)] - r
        rowid = j * L + lane
        for k in range(MAX_REP):
            gpos = off + k
            pos = gpos - base
            mask = (r > k) & (pos >= 0) & (pos < ROWS_PER_W) & (gpos < limit)
            plsc.store_scatter(idx_v, [pos], rowid, mask=mask)
        return carry

    lax.fori_loop(0, IN_ROWS // L, scan, 0)
    pltpu.sync_copy(idx_v, idx_hbm.at[pl.ds(base, ROWS_PER_W)])


_sc_idx = functools.partial(
    pl.kernel,
    out_type=jax.ShapeDtypeStruct((OUT_ROWS,), jnp.int32),
    mesh=plsc.VectorSubcoreMesh(core_axis_name="c", subcore_axis_name="s"),
    compiler_params=pltpu.CompilerParams(needs_layout_passes=False),
    scratch_types=[
        pltpu.VMEM((IN_ROWS,), jnp.int32),
        pltpu.VMEM((IN_ROWS,), jnp.int32),
        pltpu.VMEM((ROWS_PER_W,), jnp.int32),
        pltpu.VMEM((L,), jnp.int32),
    ],
)(_idx_body)


def _gather_tc(src_ref, feeds_hbm, zeros_hbm, out_hbm,
               feeds_v, stage, std0, std1, state, fsem, wsem):
    step = pl.program_id(0)
    row0 = (N_BLOCKS - 1 - step) * OUT_BLOCK
    stds = (std0, std1)

    def feeds_dma():
        return pltpu.make_async_copy(feeds_hbm, feeds_v.at[pl.ds(0, IN_ROWS)],
                                     fsem)

    def wb_dma(buf, r0):
        return pltpu.make_async_copy(
            buf, out_hbm.at[pl.ds(r0, OUT_BLOCK)], wsem)

    @pl.when(step == 0)
    def _():
        feeds_dma().start()
        feeds_v[IN_ROWS, :, :] = jnp.zeros((8, 128), jnp.float32)
        state[0] = 0
        state[1] = 0

    @pl.when(src_ref[0] != IN_ROWS)
    def _():
        @pl.when(state[0] == 0)
        def _():
            feeds_dma().wait()
            state[0] = 1

        def copy_row(i, carry):
            s = src_ref[i]
            stage[pl.ds(i, 1)] = feeds_v[pl.ds(s, 1)]
            return carry

        lax.fori_loop(0, OUT_BLOCK, copy_row, 0, unroll=16)
        n = state[1]

        @pl.when(n % 2 == 0)
        def _():
            std0[...] = stage[...].reshape(OUT_BLOCK, D)

        @pl.when(n % 2 == 1)
        def _():
            std1[...] = stage[...].reshape(OUT_BLOCK, D)

        @pl.when(n > 0)
        def _():
            wb_dma(std0, row0).wait()

        @pl.when(n % 2 == 0)
        def _():
            wb_dma(std0, row0).start()

        @pl.when(n % 2 == 1)
        def _():
            wb_dma(std1, row0).start()

        state[1] = n + 1

    @pl.when(step == N_BLOCKS - 1)
    def _():
        @pl.when(state[0] == 0)
        def _():
            feeds_dma().wait()
            state[0] = 1

        @pl.when(state[1] > 0)
        def _():
            wb_dma(std0, row0).wait()


_tc_gather = pl.pallas_call(
    _gather_tc,
    grid=(N_BLOCKS,),
    in_specs=[
        pl.BlockSpec((OUT_BLOCK,), lambda g: (N_BLOCKS - 1 - g,),
                     memory_space=pltpu.SMEM),
        pl.BlockSpec(memory_space=pl.ANY),
        pl.BlockSpec(memory_space=pl.ANY),
    ],
    out_specs=pl.BlockSpec(memory_space=pl.ANY),
    out_shape=jax.ShapeDtypeStruct((OUT_ROWS, D), jnp.float32),
    scratch_shapes=[
        pltpu.VMEM((IN_ROWS + 1, 8, 128), jnp.float32),
        pltpu.VMEM((OUT_BLOCK, 8, 128), jnp.float32),
        pltpu.VMEM((OUT_BLOCK, D), jnp.float32),
        pltpu.VMEM((OUT_BLOCK, D), jnp.float32),
        pltpu.SMEM((2,), jnp.int32),
        pltpu.SemaphoreType.DMA,
        pltpu.SemaphoreType.DMA,
    ],
    input_output_aliases={2: 0},
)


def kernel(feeds, feeds_repeat_times, output_feeds_size):
    rt = feeds_repeat_times.astype(jnp.int32)
    cum = jnp.cumsum(rt)
    limit = jnp.full((L,), jnp.minimum(output_feeds_size, OUT_ROWS), jnp.int32)
    src = _sc_idx(rt, cum, limit)
    feeds_r = feeds.reshape(IN_ROWS, 8, 128)
    zeros = jnp.zeros((OUT_ROWS, D), jnp.float32)
    return _tc_gather(src, feeds_r, zeros)


# speedup vs baseline: 1.0458x; 1.0458x over previous
import functools

import jax
import jax.numpy as jnp
from jax import lax
from jax.experimental import pallas as pl
from jax.experimental.pallas import tpu as pltpu
from jax.experimental.pallas import tpu_sc as plsc

NUM_CORES = 2
NUM_SUBCORES = 16
NW = NUM_CORES * NUM_SUBCORES
L = 16

IN_ROWS = 8192
OUT_ROWS = 32768
D = 1024
ROWS_PER_W = OUT_ROWS // NW
MAX_REP = 3

OUT_BLOCK = 1024
N_BLOCKS = OUT_ROWS // OUT_BLOCK


def _idx_body(rt_hbm, cum_hbm, lim_hbm, zeros_hbm, idx_hbm,
              rt_v, cum_v, idx_v, lim_v):
    wid = lax.axis_index("s") * NUM_CORES + lax.axis_index("c")
    base = wid * ROWS_PER_W

    pltpu.sync_copy(rt_hbm, rt_v)
    pltpu.sync_copy(cum_hbm, cum_v)
    pltpu.sync_copy(lim_hbm, lim_v)
    limit = lim_v[...]

    def init(j, carry):
        idx_v[pl.ds(j * L, L)] = jnp.full((L,), IN_ROWS, jnp.int32)
        return carry

    lax.fori_loop(0, ROWS_PER_W // L, init, 0)

    lane = lax.iota(jnp.int32, L)

    def scan(j, carry):
        r = rt_v[pl.ds(j * L, L)]
        off = cum_v[pl.ds(j * L, L)] - r
        rowid = j * L + lane
        for k in range(MAX_REP):
            gpos = off + k
            pos = gpos - base
            mask = (r > k) & (pos >= 0) & (pos < ROWS_PER_W) & (gpos < limit)
            plsc.store_scatter(idx_v, [pos], rowid, mask=mask)
        return carry

    lax.fori_loop(0, IN_ROWS // L, scan, 0)
    pltpu.sync_copy(idx_v, idx_hbm.at[pl.ds(base, ROWS_PER_W)])


_sc_idx = functools.partial(
    pl.kernel,
    out_type=jax.ShapeDtypeStruct((OUT_ROWS,), jnp.int32),
    mesh=plsc.VectorSubcoreMesh(core_axis_name="c", subcore_axis_name="s"),
    compiler_params=pltpu.CompilerParams(needs_layout_passes=False),
    scratch_types=[
        pltpu.VMEM((IN_ROWS,), jnp.int32),
        pltpu.VMEM((IN_ROWS,), jnp.int32),
        pltpu.VMEM((ROWS_PER_W,), jnp.int32),
        pltpu.VMEM((L,), jnp.int32),
    ],
)(_idx_body)


def _gather_tc(src_ref, feeds_hbm, zeros_hbm, out_hbm,
               feeds_v, stage, std0, std1, state, fsem, wsem):
    step = pl.program_id(0)
    row0 = (N_BLOCKS - 1 - step) * OUT_BLOCK
    stds = (std0, std1)

    def feeds_dma():
        return pltpu.make_async_copy(feeds_hbm, feeds_v.at[pl.ds(0, IN_ROWS)],
                                     fsem)

    def wb_dma(buf, r0):
        return pltpu.make_async_copy(
            buf, out_hbm.at[pl.ds(r0, OUT_BLOCK)], wsem)

    @pl.when(step == 0)
    def _():
        feeds_dma().start()
        feeds_v[IN_ROWS, :, :] = jnp.zeros((8, 128), jnp.float32)
        state[0] = 0
        state[1] = 0

    @pl.when(src_ref[0] != IN_ROWS)
    def _():
        @pl.when(state[0] == 0)
        def _():
            feeds_dma().wait()
            state[0] = 1

        def copy_row(i, carry):
            s = src_ref[i]
            stage[pl.ds(i, 1)] = feeds_v[pl.ds(s, 1)]
            return carry

        lax.fori_loop(0, OUT_BLOCK, copy_row, 0, unroll=16)
        n = state[1]

        @pl.when(n % 2 == 0)
        def _():
            std0[...] = stage[...].reshape(OUT_BLOCK, D)

        @pl.when(n % 2 == 1)
        def _():
            std1[...] = stage[...].reshape(OUT_BLOCK, D)

        @pl.when(n > 0)
        def _():
            wb_dma(std0, row0).wait()

        @pl.when(n % 2 == 0)
        def _():
            wb_dma(std0, row0).start()

        @pl.when(n % 2 == 1)
        def _():
            wb_dma(std1, row0).start()

        state[1] = n + 1

    @pl.when(step == N_BLOCKS - 1)
    def _():
        @pl.when(state[0] == 0)
        def _():
            feeds_dma().wait()
            state[0] = 1

        @pl.when(state[1] > 0)
        def _():
            wb_dma(std0, row0).wait()


_tc_gather = pl.pallas_call(
    _gather_tc,
    grid=(N_BLOCKS,),
    in_specs=[
        pl.BlockSpec((OUT_BLOCK,), lambda g: (N_BLOCKS - 1 - g,),
                     memory_space=pltpu.SMEM),
        pl.BlockSpec(memory_space=pl.ANY),
        pl.BlockSpec(memory_space=pl.ANY),
    ],
    out_specs=pl.BlockSpec(memory_space=pl.ANY),
    out_shape=jax.ShapeDtypeStruct((OUT_ROWS, D), jnp.float32),
    scratch_shapes=[
        pltpu.VMEM((IN_ROWS + 1, 8, 128), jnp.float32),
        pltpu.VMEM((OUT_BLOCK, 8, 128), jnp.float32),
        pltpu.VMEM((OUT_BLOCK, D), jnp.float32),
        pltpu.VMEM((OUT_BLOCK, D), jnp.float32),
        pltpu.SMEM((2,), jnp.int32),
        pltpu.SemaphoreType.DMA,
        pltpu.SemaphoreType.DMA,
    ],
    input_output_aliases={2: 0},
)


def kernel(feeds, feeds_repeat_times, output_feeds_size):
    rt = feeds_repeat_times.astype(jnp.int32)
    cum = jnp.cumsum(rt)
    limit = jnp.full((L,), jnp.minimum(output_feeds_size, OUT_ROWS), jnp.int32)
    zeros = jnp.zeros((OUT_ROWS, D), jnp.float32)
    src = _sc_idx(rt, cum, limit, zeros)
    feeds_r = feeds.reshape(IN_ROWS, 8, 128)
    return _tc_gather(src, feeds_r, zeros)


# speedup vs baseline: 1.2456x; 1.1911x over previous
import functools

import jax
import jax.numpy as jnp
from jax import lax
from jax.experimental import pallas as pl
from jax.experimental.pallas import tpu as pltpu
from jax.experimental.pallas import tpu_sc as plsc

NUM_CORES = 2
NUM_SUBCORES = 16
NW = NUM_CORES * NUM_SUBCORES
L = 16

IN_ROWS = 8192
OUT_ROWS = 32768
D = 1024
ROWS_PER_W = OUT_ROWS // NW
MAX_REP = 3

OUT_BLOCK = 1024
N_BLOCKS = OUT_ROWS // OUT_BLOCK


def _idx_body(rt_hbm, cum_hbm, lim_hbm, bnd_hbm, idx_hbm,
              rt_v, cum_v, idx_v, lim_v, bnd_v):
    wid = lax.axis_index("s") * NUM_CORES + lax.axis_index("c")
    base = wid * ROWS_PER_W

    pltpu.sync_copy(rt_hbm, rt_v)
    pltpu.sync_copy(cum_hbm, cum_v)
    pltpu.sync_copy(lim_hbm, lim_v)
    pltpu.sync_copy(bnd_hbm, bnd_v)
    limit = lim_v[...]
    vlo = bnd_v[pl.ds(wid, L)][0]
    vhi = bnd_v[pl.ds(NW + wid, L)][0]

    def init(j, carry):
        idx_v[pl.ds(j * L, L)] = jnp.full((L,), IN_ROWS, jnp.int32)
        return carry

    lax.fori_loop(0, ROWS_PER_W // L, init, 0)

    lane = lax.iota(jnp.int32, L)

    def scan(j, carry):
        r = rt_v[pl.ds(j * L, L)]
        off = cum_v[pl.ds(j * L, L)] - r
        rowid = j * L + lane
        for k in range(MAX_REP):
            gpos = off + k
            pos = gpos - base
            mask = (r > k) & (pos >= 0) & (pos < ROWS_PER_W) & (gpos < limit)
            plsc.store_scatter(idx_v, [pos], rowid, mask=mask)
        return carry

    lax.fori_loop(vlo, vhi, scan, 0)
    pltpu.sync_copy(idx_v, idx_hbm.at[pl.ds(base, ROWS_PER_W)])


_sc_idx = functools.partial(
    pl.kernel,
    out_type=jax.ShapeDtypeStruct((OUT_ROWS,), jnp.int32),
    mesh=plsc.VectorSubcoreMesh(core_axis_name="c", subcore_axis_name="s"),
    compiler_params=pltpu.CompilerParams(needs_layout_passes=False),
    scratch_types=[
        pltpu.VMEM((IN_ROWS,), jnp.int32),
        pltpu.VMEM((IN_ROWS,), jnp.int32),
        pltpu.VMEM((ROWS_PER_W,), jnp.int32),
        pltpu.VMEM((L,), jnp.int32),
        pltpu.VMEM((2 * NW + L,), jnp.int32),
    ],
)(_idx_body)


def _gather_tc(src_ref, feeds_hbm, out_ref, feeds_v, stage, done, sem):
    step = pl.program_id(0)

    def feeds_dma():
        return pltpu.make_async_copy(feeds_hbm, feeds_v.at[pl.ds(0, IN_ROWS)],
                                     sem)

    @pl.when(step == 0)
    def _():
        feeds_dma().start()
        feeds_v[IN_ROWS, :, :] = jnp.zeros((8, 128), jnp.float32)
        done[0] = 0

    all_pad = src_ref[0] == IN_ROWS

    @pl.when(all_pad)
    def _():
        out_ref[...] = jnp.zeros((OUT_BLOCK, D), jnp.float32)

    @pl.when(jnp.logical_not(all_pad))
    def _():
        @pl.when(done[0] == 0)
        def _():
            feeds_dma().wait()
            done[0] = 1

        def copy_row(i, carry):
            s = src_ref[i]
            stage[pl.ds(i, 1)] = feeds_v[pl.ds(s, 1)]
            return carry

        lax.fori_loop(0, OUT_BLOCK, copy_row, 0, unroll=16)
        out_ref[...] = stage[...].reshape(OUT_BLOCK, D)

    @pl.when(jnp.logical_and(step == N_BLOCKS - 1, done[0] == 0))
    def _():
        feeds_dma().wait()
        done[0] = 1


_tc_gather = pl.pallas_call(
    _gather_tc,
    grid=(N_BLOCKS,),
    in_specs=[
        pl.BlockSpec((OUT_BLOCK,), lambda g: (N_BLOCKS - 1 - g,),
                     memory_space=pltpu.SMEM),
        pl.BlockSpec(memory_space=pl.ANY),
    ],
    out_specs=pl.BlockSpec((OUT_BLOCK, D), lambda g: (N_BLOCKS - 1 - g, 0)),
    out_shape=jax.ShapeDtypeStruct((OUT_ROWS, D), jnp.float32),
    scratch_shapes=[
        pltpu.VMEM((IN_ROWS + 1, 8, 128), jnp.float32),
        pltpu.VMEM((OUT_BLOCK, 8, 128), jnp.float32),
        pltpu.SMEM((1,), jnp.int32),
        pltpu.SemaphoreType.DMA,
    ],
)


def kernel(feeds, feeds_repeat_times, output_feeds_size):
    rt = feeds_repeat_times.astype(jnp.int32)
    cum = jnp.cumsum(rt)
    limit = jnp.full((L,), jnp.minimum(output_feeds_size, OUT_ROWS), jnp.int32)
    starts = jnp.arange(NW, dtype=jnp.int32) * ROWS_PER_W
    lo = jnp.searchsorted(cum, starts, side="right").astype(jnp.int32)
    hi = jnp.searchsorted(cum - rt, starts + ROWS_PER_W,
                          side="left").astype(jnp.int32)
    bounds = jnp.concatenate(
        [lo // L, (hi + L - 1) // L, jnp.zeros((L,), jnp.int32)])
    src = _sc_idx(rt, cum, limit, bounds)
    feeds_r = feeds.reshape(IN_ROWS, 8, 128)
    return _tc_gather(src, feeds_r)
